# baseline (speedup 1.0000x reference)
import jax
import jax.numpy as jnp
from jax.experimental import pallas as pl
from jax.experimental.pallas import tpu as pltpu

_TB = 512


def _fused_body(ids_ref, emb_ref, w1t_ref, b1_ref, w2t_ref, b2_ref,
                x_ref, h1_ref, h2_ref):
    ids = ids_ref[...]
    bdim, tdim = ids.shape
    iota = jax.lax.broadcasted_iota(jnp.int32, (bdim, tdim, 128), 2)
    onehot = (ids[:, :, None] == iota).astype(jnp.float32)
    dn = (((2,), (0,)), ((), ()))
    x = jax.lax.dot_general(onehot, emb_ref[...], dn,
                            preferred_element_type=jnp.float32)
    x_ref[...] = x
    b1 = b1_ref[0][None, None, :]
    h1 = jax.lax.dot_general(x, w1t_ref[...], dn,
                             preferred_element_type=jnp.float32) + b1
    h1_ref[...] = h1
    b2 = b2_ref[0][None, None, :]
    h2 = jax.lax.dot_general(h1, w2t_ref[...], dn,
                             preferred_element_type=jnp.float32) + b2
    h2_ref[...] = h2


def kernel(input_ids, emb, W1, b1, W2, b2):
    B, S = input_ids.shape
    V, H = emb.shape
    embp = jnp.zeros((128, H), dtype=emb.dtype).at[:V].set(emb)
    w1t = W1.T
    w2t = W2.T
    b1r = b1.reshape(1, H)
    b2r = b2.reshape(1, H)

    nblk = S // _TB
    full = lambda i: (0, 0)
    grid_spec = pl.GridSpec(
        grid=(nblk,),
        in_specs=[
            pl.BlockSpec((B, _TB), lambda i: (0, i)),
            pl.BlockSpec((128, H), full),
            pl.BlockSpec((H, H), full),
            pl.BlockSpec((1, H), full),
            pl.BlockSpec((H, H), full),
            pl.BlockSpec((1, H), full),
        ],
        out_specs=[
            pl.BlockSpec((B, _TB, H), lambda i: (0, i, 0)),
            pl.BlockSpec((B, _TB, H), lambda i: (0, i, 0)),
            pl.BlockSpec((B, _TB, H), lambda i: (0, i, 0)),
        ],
    )
    out_shape = [jax.ShapeDtypeStruct((B, S, H), jnp.float32)] * 3
    x, h1, h2 = pl.pallas_call(
        _fused_body,
        grid_spec=grid_spec,
        out_shape=out_shape,
        compiler_params=pltpu.CompilerParams(
            dimension_semantics=("arbitrary",),
        ),
    )(input_ids, embp, w1t, b1r, w2t, b2r)
    return (x, h1, h2)


# speedup vs baseline: 1.1365x; 1.1365x over previous
import jax
import jax.numpy as jnp
from jax.experimental import pallas as pl
from jax.experimental.pallas import tpu as pltpu

_TB = 1024


def _fused_body(ids_ref, emb_ref, w1t_ref, b1_ref, w2t_ref, b2_ref,
                x_ref, h1_ref, h2_ref):
    ids = ids_ref[...]
    bdim, tdim = ids.shape
    iota = jax.lax.broadcasted_iota(jnp.int32, (bdim, tdim, 128), 2)
    onehot = (ids[:, :, None] == iota).astype(jnp.float32)
    dn = (((2,), (0,)), ((), ()))
    x = jax.lax.dot_general(onehot, emb_ref[...], dn,
                            preferred_element_type=jnp.float32)
    x_ref[...] = x
    b1 = b1_ref[0][None, None, :]
    h1 = jax.lax.dot_general(x, w1t_ref[...], dn,
                             preferred_element_type=jnp.float32) + b1
    h1_ref[...] = h1
    b2 = b2_ref[0][None, None, :]
    h2 = jax.lax.dot_general(h1, w2t_ref[...], dn,
                             preferred_element_type=jnp.float32) + b2
    h2_ref[...] = h2


def kernel(input_ids, emb, W1, b1, W2, b2):
    B, S = input_ids.shape
    V, H = emb.shape
    embp = jnp.zeros((128, H), dtype=emb.dtype).at[:V].set(emb)
    w1t = W1.T
    w2t = W2.T
    b1r = b1.reshape(1, H)
    b2r = b2.reshape(1, H)

    nblk = S // _TB
    full = lambda i: (0, 0)
    grid_spec = pl.GridSpec(
        grid=(nblk,),
        in_specs=[
            pl.BlockSpec((B, _TB), lambda i: (0, i)),
            pl.BlockSpec((128, H), full),
            pl.BlockSpec((H, H), full),
            pl.BlockSpec((1, H), full),
            pl.BlockSpec((H, H), full),
            pl.BlockSpec((1, H), full),
        ],
        out_specs=[
            pl.BlockSpec((B, _TB, H), lambda i: (0, i, 0)),
            pl.BlockSpec((B, _TB, H), lambda i: (0, i, 0)),
            pl.BlockSpec((B, _TB, H), lambda i: (0, i, 0)),
        ],
    )
    out_shape = [jax.ShapeDtypeStruct((B, S, H), jnp.float32)] * 3
    x, h1, h2 = pl.pallas_call(
        _fused_body,
        grid_spec=grid_spec,
        out_shape=out_shape,
        compiler_params=pltpu.CompilerParams(
            dimension_semantics=("arbitrary",),
        ),
    )(input_ids, embp, w1t, b1r, w2t, b2r)
    return (x, h1, h2)
